# Initial kernel scaffold; baseline (speedup 1.0000x reference)
import jax
import jax.numpy as jnp
from jax import lax
from jax.experimental import pallas as pl
from jax.experimental.pallas import tpu as pltpu
from jax.experimental.pallas import tpu_sc as plsc

NC = 2
NS = 16
NW = NC * NS
L = 16

E = 320000
D = 128
EPW = E // NW
C = 80
NCHUNK = EPW // C
NGRP = C // L


def _sc_body(zu_hbm, zi_hbm, ui_hbm, vi_hbm, out_hbm,
             uidx, vidx, urows, vrows, outv, sem_u, sem_v):
    wid = lax.axis_index("s") * NC + lax.axis_index("c")
    base = wid * EPW

    pltpu.sync_copy(ui_hbm.at[pl.ds(base, EPW)], uidx)
    pltpu.sync_copy(vi_hbm.at[pl.ds(base, EPW)], vidx)

    lane = lax.broadcasted_iota(jnp.int32, (L,), 0)
    one = jnp.float32(1.0)

    def chunk(g, carry):
        cbase = g * C
        cu = pltpu.async_copy(zu_hbm.at[uidx.at[pl.ds(cbase, C)]], urows, sem_u)
        cv = pltpu.async_copy(zi_hbm.at[vidx.at[pl.ds(cbase, C)]], vrows, sem_v)
        cu.wait()
        cv.wait()

        def grp(gg, c2):
            rows = lane + gg * L
            acc = jnp.zeros((L,), jnp.float32)
            for d in range(D):
                dvec = jnp.full((L,), d, jnp.int32)
                u = plsc.load_gather(urows, [rows, dvec])
                v = plsc.load_gather(vrows, [rows, dvec])
                acc = acc + u * v
            s = one / (one + jnp.exp(-acc))
            outv[pl.ds(cbase + gg * L, L)] = s
            return c2

        return lax.fori_loop(0, NGRP, grp, carry)

    lax.fori_loop(0, NCHUNK, chunk, 0)

    pltpu.sync_copy(outv, out_hbm.at[pl.ds(base, EPW)])


@jax.jit
def _decode(z_user, z_item, ui, vi):
    mesh = plsc.VectorSubcoreMesh(core_axis_name="c", subcore_axis_name="s")
    return pl.kernel(
        _sc_body,
        out_type=jax.ShapeDtypeStruct((E,), jnp.float32),
        mesh=mesh,
        scratch_types=[
            pltpu.VMEM((EPW,), jnp.int32),
            pltpu.VMEM((EPW,), jnp.int32),
            pltpu.VMEM((C, D), jnp.float32),
            pltpu.VMEM((C, D), jnp.float32),
            pltpu.VMEM((EPW,), jnp.float32),
            pltpu.SemaphoreType.DMA,
            pltpu.SemaphoreType.DMA,
        ],
    )(z_user, z_item, ui, vi)


def kernel(z_user, z_item, edge_index):
    return _decode(z_user, z_item, edge_index[0], edge_index[1])


# speedup vs baseline: 1.1864x; 1.1864x over previous
import jax
import jax.numpy as jnp
from jax import lax
from jax.experimental import pallas as pl
from jax.experimental.pallas import tpu as pltpu
from jax.experimental.pallas import tpu_sc as plsc

NC = 2
NS = 16
NW = NC * NS
L = 16

E = 320000
D = 128
EPW = E // NW
C = 80
NCHUNK = EPW // C
NGRP = C // L


def _sc_body(zu_hbm, zi_hbm, ui_hbm, vi_hbm, out_hbm,
             uidx, vidx, urows, vrows, outv, sem_u, sem_v):
    wid = lax.axis_index("s") * NC + lax.axis_index("c")
    base = wid * EPW

    pltpu.sync_copy(ui_hbm.at[pl.ds(base, EPW)], uidx)
    pltpu.sync_copy(vi_hbm.at[pl.ds(base, EPW)], vidx)

    lane = lax.broadcasted_iota(jnp.int32, (L,), 0)
    one = jnp.float32(1.0)

    def chunk(g, carry):
        cbase = g * C
        cu = pltpu.async_copy(zu_hbm.at[uidx.at[pl.ds(cbase, C)]], urows, sem_u)
        cv = pltpu.async_copy(zi_hbm.at[vidx.at[pl.ds(cbase, C)]], vrows, sem_v)
        cu.wait()
        cv.wait()

        def grp(gg, c2):
            rows = lane + gg * L
            acc = jnp.zeros((L,), jnp.float32)
            for d in range(D):
                dvec = jnp.full((L,), d, jnp.int32)
                u = plsc.load_gather(urows, [rows, dvec])
                v = plsc.load_gather(vrows, [rows, dvec])
                acc = acc + u * v
            s = one / (one + jnp.exp(-acc))
            outv[pl.ds(cbase + gg * L, L)] = s
            return c2

        return lax.fori_loop(0, NGRP, grp, carry)

    lax.fori_loop(0, NCHUNK, chunk, 0)

    pltpu.sync_copy(outv, out_hbm.at[pl.ds(base, EPW)])


@jax.jit
def _decode(z_user, z_item, ui, vi):
    mesh = plsc.VectorSubcoreMesh(core_axis_name="c", subcore_axis_name="s")
    return pl.kernel(
        _sc_body,
        out_type=jax.ShapeDtypeStruct((E,), jnp.float32),
        mesh=mesh,
        compiler_params=pltpu.CompilerParams(needs_layout_passes=False),
        scratch_types=[
            pltpu.VMEM((EPW,), jnp.int32),
            pltpu.VMEM((EPW,), jnp.int32),
            pltpu.VMEM((C, D), jnp.float32),
            pltpu.VMEM((C, D), jnp.float32),
            pltpu.VMEM((EPW,), jnp.float32),
            pltpu.SemaphoreType.DMA,
            pltpu.SemaphoreType.DMA,
        ],
    )(z_user, z_item, ui, vi)


def kernel(z_user, z_item, edge_index):
    return _decode(z_user, z_item, edge_index[0], edge_index[1])


# speedup vs baseline: 1.3332x; 1.1237x over previous
import jax
import jax.numpy as jnp
from jax import lax
from jax.experimental import pallas as pl
from jax.experimental.pallas import tpu as pltpu
from jax.experimental.pallas import tpu_sc as plsc

NC = 2
NS = 16
NW = NC * NS
L = 16

E = 320000
D = 128
EPW = E // NW
C = 80
NCHUNK = EPW // C
NGRP = C // L


def _sc_body(zu_hbm, zi_hbm, ui_hbm, vi_hbm, out_hbm,
             uidx, vidx, urows0, urows1, vrows0, vrows1, outv,
             sem_u0, sem_u1, sem_v0, sem_v1):
    wid = lax.axis_index("s") * NC + lax.axis_index("c")
    base = wid * EPW

    pltpu.sync_copy(ui_hbm.at[pl.ds(base, EPW)], uidx)
    pltpu.sync_copy(vi_hbm.at[pl.ds(base, EPW)], vidx)

    ubufs = (urows0, urows1)
    vbufs = (vrows0, vrows1)
    usems = (sem_u0, sem_u1)
    vsems = (sem_v0, sem_v1)

    lane = lax.broadcasted_iota(jnp.int32, (L,), 0)
    one = jnp.float32(1.0)

    def start(cb, b):
        pltpu.async_copy(zu_hbm.at[uidx.at[pl.ds(cb * C, C)]], ubufs[b], usems[b])
        pltpu.async_copy(zi_hbm.at[vidx.at[pl.ds(cb * C, C)]], vbufs[b], vsems[b])

    def wait(cb, b):
        pltpu.make_async_copy(
            zu_hbm.at[uidx.at[pl.ds(cb * C, C)]], ubufs[b], usems[b]).wait()
        pltpu.make_async_copy(
            zi_hbm.at[vidx.at[pl.ds(cb * C, C)]], vbufs[b], vsems[b]).wait()

    def compute(cb, b):
        ur, vr = ubufs[b], vbufs[b]

        def grp(gg, c2):
            rows = lane + gg * L
            acc = jnp.zeros((L,), jnp.float32)
            for d in range(D):
                dvec = jnp.full((L,), d, jnp.int32)
                u = plsc.load_gather(ur, [rows, dvec])
                v = plsc.load_gather(vr, [rows, dvec])
                acc = acc + u * v
            s = one / (one + jnp.exp(-acc))
            outv[pl.ds(cb * C + gg * L, L)] = s
            return c2

        lax.fori_loop(0, NGRP, grp, 0)

    start(0, 0)
    start(1, 1)

    def body(g, carry):
        for b in range(2):
            cb = 2 * g + b
            wait(cb, b)
            compute(cb, b)
            start(cb + 2, b)
        return carry

    lax.fori_loop(0, (NCHUNK - 3) // 2, body, 0)

    wait(NCHUNK - 3, 0)
    compute(NCHUNK - 3, 0)
    start(NCHUNK - 1, 0)
    wait(NCHUNK - 2, 1)
    compute(NCHUNK - 2, 1)
    wait(NCHUNK - 1, 0)
    compute(NCHUNK - 1, 0)

    pltpu.sync_copy(outv, out_hbm.at[pl.ds(base, EPW)])


@jax.jit
def _decode(z_user, z_item, ui, vi):
    mesh = plsc.VectorSubcoreMesh(core_axis_name="c", subcore_axis_name="s")
    return pl.kernel(
        _sc_body,
        out_type=jax.ShapeDtypeStruct((E,), jnp.float32),
        mesh=mesh,
        compiler_params=pltpu.CompilerParams(needs_layout_passes=False),
        scratch_types=[
            pltpu.VMEM((EPW,), jnp.int32),
            pltpu.VMEM((EPW,), jnp.int32),
            pltpu.VMEM((C, D), jnp.float32),
            pltpu.VMEM((C, D), jnp.float32),
            pltpu.VMEM((C, D), jnp.float32),
            pltpu.VMEM((C, D), jnp.float32),
            pltpu.VMEM((EPW,), jnp.float32),
            pltpu.SemaphoreType.DMA,
            pltpu.SemaphoreType.DMA,
            pltpu.SemaphoreType.DMA,
            pltpu.SemaphoreType.DMA,
        ],
    )(z_user, z_item, ui, vi)


def kernel(z_user, z_item, edge_index):
    return _decode(z_user, z_item, edge_index[0], edge_index[1])


# speedup vs baseline: 8.0358x; 6.0276x over previous
import jax
import jax.numpy as jnp
from jax import lax
from jax.experimental import pallas as pl
from jax.experimental.pallas import tpu as pltpu
from jax.experimental.pallas import tpu_sc as plsc

NC = 2
NS = 16
NW = NC * NS
L = 16

E = 320000
D = 128
EPW = E // NW
C = 80
NCHUNK = EPW // C
NGRP = C // L


def _sc_body(zu_hbm, zi_hbm, ui_hbm, vi_hbm, out_hbm,
             uidx, vidx, urows0, urows1, vrows0, vrows1, outv, pbuf,
             sem_u0, sem_u1, sem_v0, sem_v1):
    wid = lax.axis_index("s") * NC + lax.axis_index("c")
    base = wid * EPW

    pltpu.sync_copy(ui_hbm.at[pl.ds(base, EPW)], uidx)
    pltpu.sync_copy(vi_hbm.at[pl.ds(base, EPW)], vidx)

    ubufs = (urows0, urows1)
    vbufs = (vrows0, vrows1)
    usems = (sem_u0, sem_u1)
    vsems = (sem_v0, sem_v1)

    lane = lax.broadcasted_iota(jnp.int32, (L,), 0)
    one = jnp.float32(1.0)

    def start(cb, b):
        pltpu.async_copy(zu_hbm.at[uidx.at[pl.ds(cb * C, C)]], ubufs[b], usems[b])
        pltpu.async_copy(zi_hbm.at[vidx.at[pl.ds(cb * C, C)]], vbufs[b], vsems[b])

    def wait(cb, b):
        pltpu.make_async_copy(
            zu_hbm.at[uidx.at[pl.ds(cb * C, C)]], ubufs[b], usems[b]).wait()
        pltpu.make_async_copy(
            zi_hbm.at[vidx.at[pl.ds(cb * C, C)]], vbufs[b], vsems[b]).wait()

    lane17 = lane * 17

    def compute(cb, b):
        ur, vr = ubufs[b], vbufs[b]

        def grp(gg, c2):
            ebase = gg * L
            for e in range(L):
                re = ebase + e
                acc = ur[re, pl.ds(0, L)] * vr[re, pl.ds(0, L)]
                for j in range(1, D // L):
                    acc = acc + ur[re, pl.ds(j * L, L)] * vr[re, pl.ds(j * L, L)]
                pbuf[pl.ds(17 * e, L)] = acc
            tot = plsc.load_gather(pbuf, [lane17])
            for l in range(1, L):
                tot = tot + plsc.load_gather(pbuf, [lane17 + l])
            s = one / (one + jnp.exp(-tot))
            outv[pl.ds(cb * C + ebase, L)] = s
            return c2

        lax.fori_loop(0, NGRP, grp, 0)

    start(0, 0)
    start(1, 1)

    def body(g, carry):
        for b in range(2):
            cb = 2 * g + b
            wait(cb, b)
            compute(cb, b)
            start(cb + 2, b)
        return carry

    lax.fori_loop(0, (NCHUNK - 3) // 2, body, 0)

    wait(NCHUNK - 3, 0)
    compute(NCHUNK - 3, 0)
    start(NCHUNK - 1, 0)
    wait(NCHUNK - 2, 1)
    compute(NCHUNK - 2, 1)
    wait(NCHUNK - 1, 0)
    compute(NCHUNK - 1, 0)

    pltpu.sync_copy(outv, out_hbm.at[pl.ds(base, EPW)])


@jax.jit
def _decode(z_user, z_item, ui, vi):
    mesh = plsc.VectorSubcoreMesh(core_axis_name="c", subcore_axis_name="s")
    return pl.kernel(
        _sc_body,
        out_type=jax.ShapeDtypeStruct((E,), jnp.float32),
        mesh=mesh,
        compiler_params=pltpu.CompilerParams(needs_layout_passes=False),
        scratch_types=[
            pltpu.VMEM((EPW,), jnp.int32),
            pltpu.VMEM((EPW,), jnp.int32),
            pltpu.VMEM((C, D), jnp.float32),
            pltpu.VMEM((C, D), jnp.float32),
            pltpu.VMEM((C, D), jnp.float32),
            pltpu.VMEM((C, D), jnp.float32),
            pltpu.VMEM((EPW,), jnp.float32),
            pltpu.VMEM((L * 17,), jnp.float32),
            pltpu.SemaphoreType.DMA,
            pltpu.SemaphoreType.DMA,
            pltpu.SemaphoreType.DMA,
            pltpu.SemaphoreType.DMA,
        ],
    )(z_user, z_item, ui, vi)


def kernel(z_user, z_item, edge_index):
    return _decode(z_user, z_item, edge_index[0], edge_index[1])
